# baseline (speedup 1.0000x reference)
import jax
import jax.numpy as jnp
from jax.experimental import pallas as pl
from jax.experimental.pallas import tpu as pltpu
from jax.experimental.pallas import tpu_sc as plsc

K = 9
C = 128
COUT = 128
MAX_DEPTH = 8

_GATHER_WINDOW = 128
_GATHER_SPLIT = 8
_BLK = 2000


def _sc_gather(xp, flat_idx):
    num_idx = flat_idx.shape[1]
    mesh = plsc.VectorSubcoreMesh(core_axis_name="core", subcore_axis_name="subcore")

    @pl.kernel(
        out_type=jax.ShapeDtypeStruct((num_idx, xp.shape[1]), xp.dtype),
        mesh=mesh,
        scratch_types=[pltpu.SemaphoreType.DMA((_GATHER_SPLIT,))],
    )
    def gather_kernel(x_hbm, i_hbm, o_hbm, sems):
        sub = _GATHER_WINDOW // _GATHER_SPLIT

        def body(i_vmem, o_vmem):
            pltpu.sync_copy(x_hbm.at[i_vmem.at[0]], o_vmem)

        pltpu.emit_pipeline(
            body,
            grid=(num_idx // _GATHER_WINDOW,),
            in_specs=[pl.BlockSpec((1, _GATHER_WINDOW), lambda i: (0, i))],
            out_specs=[pl.BlockSpec((_GATHER_WINDOW, xp.shape[1]), lambda i: (i, 0))],
            core_axis_name="subcore",
            dimension_semantics=(pltpu.PARALLEL,),
        )(i_hbm, o_hbm)

    return gather_kernel(xp, flat_idx)


def _tc_combine(gathered, neigh_idx, neigh_depth, depth_f, wf, wd, b2, n):
    nblk = n // _BLK

    def body(dref, nid_ref, ndep_ref, wf_ref, wd_ref, b_ref, *rest):
        g_refs = rest[:K]
        o_ref = rest[K]
        acc = jax.lax.dot_general(
            g_refs[0][...].astype(jnp.bfloat16), wf_ref[0],
            (((1,), (0,)), ((), ())),
            preferred_element_type=jnp.float32,
        )
        for k in range(1, K):
            acc += jax.lax.dot_general(
                g_refs[k][...].astype(jnp.bfloat16), wf_ref[k],
                (((1,), (0,)), ((), ())),
                preferred_element_type=jnp.float32,
            )
        valid = nid_ref[...] >= 0
        dd = (dref[0, 0] - ndep_ref[...].astype(jnp.float32)) * (1.0 / float(max(MAX_DEPTH, 1)))
        dd = jnp.where(valid, dd, 0.0)
        for k in range(K):
            acc += dd[:, k:k + 1] * wd_ref[k:k + 1, :]
        o_ref[...] = acc + b_ref[...]

    grid = (nblk,)
    in_specs = [
        pl.BlockSpec(memory_space=pltpu.SMEM),
        pl.BlockSpec((_BLK, K), lambda i: (i, 0)),
        pl.BlockSpec((_BLK, K), lambda i: (i, 0)),
        pl.BlockSpec((K, C, COUT), lambda i: (0, 0, 0)),
        pl.BlockSpec((K, COUT), lambda i: (0, 0)),
        pl.BlockSpec((1, COUT), lambda i: (0, 0)),
    ] + [
        pl.BlockSpec((_BLK, C), lambda i, k=k: (k * nblk + i, 0))
        for k in range(K)
    ]
    return pl.pallas_call(
        body,
        grid=grid,
        in_specs=in_specs,
        out_specs=pl.BlockSpec((_BLK, COUT), lambda i: (i, 0)),
        out_shape=jax.ShapeDtypeStruct((n, COUT), jnp.float32),
    )(depth_f, neigh_idx, neigh_depth, wf, wd, b2, *([gathered] * K))


def kernel(x, neigh_idx, neigh_depth, depth, W, b):
    n, c = x.shape
    xp = jnp.concatenate([x, jnp.zeros((16, c), x.dtype)], axis=0)
    safe_idx = jnp.where(neigh_idx >= 0, neigh_idx, n).astype(jnp.int32) & 4095
    flat_idx = safe_idx.T.reshape(1, n * K)
    per = _GATHER_WINDOW * 32
    pad = (-flat_idx.shape[1]) % per
    if pad:
        flat_idx = jnp.concatenate(
            [flat_idx, jnp.zeros((1, pad), jnp.int32)], axis=1)

    gathered = _sc_gather(xp, flat_idx)

    wr = W.reshape(K, c + 1, COUT)
    wf = wr[:, :c, :].astype(jnp.bfloat16)
    wd = wr[:, c, :]
    depth_f = jnp.asarray(depth, jnp.float32).reshape(1, 1)
    b2 = b.reshape(1, COUT)
    return _tc_combine(gathered, neigh_idx, neigh_depth, depth_f, wf, wd, b2, n)


# speedup vs baseline: 12.6425x; 12.6425x over previous
import jax
import jax.numpy as jnp
from jax.experimental import pallas as pl
from jax.experimental.pallas import tpu as pltpu
from jax.experimental.pallas import tpu_sc as plsc

K = 9
C = 128
COUT = 128
MAX_DEPTH = 8

_GATHER_WINDOW = 128
_ZPAD = 2048
_GATHER_SPLIT = 8
_BLK = 2000


def _sc_gather(xp, flat_idx):
    num_idx = flat_idx.shape[1]
    mesh = plsc.VectorSubcoreMesh(core_axis_name="core", subcore_axis_name="subcore")

    @pl.kernel(
        out_type=jax.ShapeDtypeStruct((num_idx, xp.shape[1]), xp.dtype),
        mesh=mesh,
        scratch_types=[pltpu.SemaphoreType.DMA((_GATHER_SPLIT,))],
    )
    def gather_kernel(x_hbm, i_hbm, o_hbm, sems):
        sub = _GATHER_WINDOW // _GATHER_SPLIT

        def body(i_vmem, o_vmem):
            pltpu.sync_copy(x_hbm.at[i_vmem.at[0]], o_vmem)

        pltpu.emit_pipeline(
            body,
            grid=(num_idx // _GATHER_WINDOW,),
            in_specs=[pl.BlockSpec((1, _GATHER_WINDOW), lambda i: (0, i))],
            out_specs=[pl.BlockSpec((_GATHER_WINDOW, xp.shape[1]), lambda i: (i, 0))],
            core_axis_name=("core", "subcore"),
            dimension_semantics=(pltpu.PARALLEL,),
        )(i_hbm, o_hbm)

    return gather_kernel(xp, flat_idx)


def _tc_combine(gathered, neigh_idx, neigh_depth, depth_f, wf, wd, b2, n):
    nblk = n // _BLK

    def body(dref, nid_ref, ndep_ref, wf_ref, wd_ref, b_ref, *rest):
        g_refs = rest[:K]
        o_ref = rest[K]
        acc = jax.lax.dot_general(
            g_refs[0][...].astype(jnp.bfloat16), wf_ref[0],
            (((1,), (0,)), ((), ())),
            preferred_element_type=jnp.float32,
        )
        for k in range(1, K):
            acc += jax.lax.dot_general(
                g_refs[k][...].astype(jnp.bfloat16), wf_ref[k],
                (((1,), (0,)), ((), ())),
                preferred_element_type=jnp.float32,
            )
        valid = nid_ref[...] >= 0
        dd = (dref[0, 0] - ndep_ref[...].astype(jnp.float32)) * (1.0 / float(max(MAX_DEPTH, 1)))
        dd = jnp.where(valid, dd, 0.0)
        for k in range(K):
            acc += dd[:, k:k + 1] * wd_ref[k:k + 1, :]
        o_ref[...] = acc + b_ref[...]

    grid = (nblk,)
    in_specs = [
        pl.BlockSpec(memory_space=pltpu.SMEM),
        pl.BlockSpec((_BLK, K), lambda i: (i, 0)),
        pl.BlockSpec((_BLK, K), lambda i: (i, 0)),
        pl.BlockSpec((K, C, COUT), lambda i: (0, 0, 0)),
        pl.BlockSpec((K, COUT), lambda i: (0, 0)),
        pl.BlockSpec((1, COUT), lambda i: (0, 0)),
    ] + [
        pl.BlockSpec((_BLK, C), lambda i, k=k: (k * nblk + i, 0))
        for k in range(K)
    ]
    return pl.pallas_call(
        body,
        grid=grid,
        in_specs=in_specs,
        out_specs=pl.BlockSpec((_BLK, COUT), lambda i: (i, 0)),
        out_shape=jax.ShapeDtypeStruct((n, COUT), jnp.float32),
    )(depth_f, neigh_idx, neigh_depth, wf, wd, b2, *([gathered] * K))


def kernel(x, neigh_idx, neigh_depth, depth, W, b):
    n, c = x.shape
    xp = jnp.concatenate([x, jnp.zeros((_ZPAD, c), x.dtype)], axis=0)
    spread = n + (jnp.arange(n * K, dtype=jnp.int32).reshape(K, n).T % _ZPAD)
    safe_idx = jnp.where(neigh_idx >= 0, neigh_idx, spread).astype(jnp.int32)
    flat_idx = safe_idx.T.reshape(1, n * K)
    per = _GATHER_WINDOW * 32
    pad = (-flat_idx.shape[1]) % per
    if pad:
        pad_tgt = n + (jnp.arange(pad, dtype=jnp.int32).reshape(1, pad) % _ZPAD)
        flat_idx = jnp.concatenate([flat_idx, pad_tgt], axis=1)

    gathered = _sc_gather(xp, flat_idx)

    wr = W.reshape(K, c + 1, COUT)
    wf = wr[:, :c, :].astype(jnp.bfloat16)
    wd = wr[:, c, :]
    depth_f = jnp.asarray(depth, jnp.float32).reshape(1, 1)
    b2 = b.reshape(1, COUT)
    return _tc_combine(gathered, neigh_idx, neigh_depth, depth_f, wf, wd, b2, n)
